# baseline (speedup 1.0000x reference)
import functools

import jax
import jax.numpy as jnp
from jax import lax
from jax.experimental import pallas as pl
from jax.experimental.pallas import tpu as pltpu
from jax.experimental.pallas import tpu_sc as plsc

N_E = 8192
E_DIM = 64
BETA = 0.25
RADIAL_BINS = 16
ANGULAR_BINS = N_E // RADIAL_BINS
MAX_RADIUS = 1.1
BLK = 3072
A_PAD = ANGULAR_BINS + 8
IDX_CHUNK = 128


def _acosh(x):
    return jnp.log(x + jnp.sqrt((x - 1.0) * (x + 1.0)))


def _vq_kernel(x_ref, rc_ref, a_ref,
               comb_ref, hist_ref, tad_ref,
               loss_ref, emean_ref, div_ref, perp_ref, tab_ref, *, nblk, n):
    i = pl.program_id(0)
    x = x_ref[...]

    dotm = jax.lax.dot_general(a_ref[...], x, (((1,), (1,)), ((), ())),
                               precision=jax.lax.Precision.HIGHEST,
                               preferred_element_type=jnp.float32)
    u_t = dotm[ANGULAR_BINS:ANGULAR_BINS + 1, :]
    r = _acosh(jnp.maximum(u_t, 1.01))
    dot_v = dotm[:ANGULAR_BINS]

    iota_w = jax.lax.broadcasted_iota(jnp.int32, (ANGULAR_BINS, BLK), 0)
    dval = jnp.max(dot_v, axis=0, keepdims=True)
    best_widx = jnp.min(jnp.where(dot_v == dval, iota_w, ANGULAR_BINS),
                        axis=0, keepdims=True)

    rc = jnp.clip(rc_ref[...], 0.01, MAX_RADIUS)
    dist_r = jnp.abs(r - rc)
    iota_r = jax.lax.broadcasted_iota(jnp.int32, (RADIAL_BINS, BLK), 0)
    r_sel = []
    for _ in range(3):
        v = jnp.min(dist_r, axis=0, keepdims=True)
        idx = jnp.min(jnp.where(dist_r == v, iota_r, RADIAL_BINS),
                      axis=0, keepdims=True)
        rc_v = jnp.max(jnp.where(iota_r == idx, rc + 0.0 * dist_r, -jnp.inf),
                       axis=0, keepdims=True)
        r_sel.append((rc_v, idx))
        dist_r = jnp.where(iota_r == idx, jnp.inf, dist_r)

    best_d = jnp.full((1, BLK), jnp.inf, dtype=jnp.float32)
    best_ridx = jnp.zeros((1, BLK), dtype=jnp.int32)
    best_cosh = jnp.ones((1, BLK), dtype=jnp.float32)
    for rc_v, ridx in r_sel:
        e = jnp.exp(rc_v)
        ch = 0.5 * (e + 1.0 / e)
        sh = 0.5 * (e - 1.0 / e)
        arg = jnp.maximum(u_t * ch - sh * dval, 1.0 + 1e-7)
        d = _acosh(arg)
        mask = d < best_d
        best_d = jnp.where(mask, d, best_d)
        best_ridx = jnp.where(mask, ridx, best_ridx)
        best_cosh = jnp.where(mask, ch, best_cosh)

    comb_ref[...] = best_ridx * ANGULAR_BINS + best_widx

    rx = _acosh(jnp.maximum(u_t, 1.0 + 1e-5))
    ry = _acosh(jnp.maximum(best_cosh, 1.0 + 1e-5))
    tad_part = jnp.sum(best_d + jnp.abs(rx - ry))

    oh_w = (iota_w == best_widx).astype(jnp.bfloat16)
    oh_r = (iota_r == best_ridx).astype(jnp.bfloat16)
    hist_part = jax.lax.dot_general(
        oh_r, oh_w, (((1,), (1,)), ((), ())),
        preferred_element_type=jnp.float32)

    @pl.when(i == 0)
    def _():
        hist_ref[...] = hist_part
        tad_ref[...] = tad_part.reshape(1, 1)
        a512 = a_ref[...][:ANGULAR_BINS]
        e0t = (jax.lax.broadcasted_iota(
            jnp.int32, (ANGULAR_BINS, E_DIM), 1) == 0)
        iota_rc = jax.lax.broadcasted_iota(
            jnp.int32, (RADIAL_BINS, 1), 0)
        for rbin in range(RADIAL_BINS):
            rc_r = jnp.max(jnp.where(iota_rc == rbin, rc, -jnp.inf))
            er = jnp.exp(rc_r)
            chr_ = 0.5 * (er + 1.0 / er)
            shr = 0.5 * (er - 1.0 / er)
            tab_ref[pl.ds(rbin * ANGULAR_BINS, ANGULAR_BINS), :] = (
                a512 * shr + jnp.where(e0t, chr_, 0.0))

    @pl.when(i > 0)
    def _():
        hist_ref[...] += hist_part
        tad_ref[...] += tad_part.reshape(1, 1)

    @pl.when(i == nblk - 1)
    def _():
        e_mean = hist_ref[...] * (1.0 / n)
        emean_ref[...] = e_mean
        div = -jnp.sum(e_mean * jnp.log(e_mean + 1e-10))
        div_ref[...] = div.reshape(1, 1)
        perp_ref[...] = jnp.exp(div).reshape(1, 1)
        loss_ref[...] = (1.0 + BETA) * tad_ref[...] * (1.0 / n)


def _sc_zq_gather(rows_per_w, n, out_shape):
    nchunk = rows_per_w // IDX_CHUNK
    t_dim = out_shape[1]
    planes_per_w = rows_per_w // t_dim
    mesh = plsc.VectorSubcoreMesh(core_axis_name="c", subcore_axis_name="s")

    @functools.partial(
        pl.kernel, mesh=mesh,
        compiler_params=pltpu.CompilerParams(use_tc_tiling_on_sc=False),
        out_type=jax.ShapeDtypeStruct(out_shape, jnp.float32),
        scratch_types=[
            pltpu.VMEM((nchunk, IDX_CHUNK), jnp.int32),
            pltpu.VMEM((rows_per_w, E_DIM), jnp.float32),
            pltpu.SemaphoreType.DMA,
        ],
    )
    def zq_gather(tab_hbm, comb_hbm, zq_hbm, idx_v, rows_v, sem):
        wid = lax.axis_index("s") * 2 + lax.axis_index("c")
        base = wid * rows_per_w
        for j in range(nchunk):
            pltpu.sync_copy(
                comb_hbm.at[pl.ds(base + j * IDX_CHUNK, IDX_CHUNK)],
                idx_v.at[j])
        copies = []
        for j in range(nchunk):
            copies.append(pltpu.async_copy(
                tab_hbm.at[idx_v.at[j]],
                rows_v.at[pl.ds(j * IDX_CHUNK, IDX_CHUNK)], sem))
        for c in copies:
            c.wait()
        for p in range(planes_per_w):
            pltpu.sync_copy(rows_v.at[pl.ds(p * t_dim, t_dim)],
                            zq_hbm.at[planes_per_w * wid + p])

    return zq_gather


def kernel(u_hyp, r_centres, angular_weight):
    shape = u_hyp.shape
    flat = u_hyp.reshape(-1, shape[-1]).astype(jnp.float32)
    n = flat.shape[0]
    nblk = n // BLK
    aw = angular_weight.astype(jnp.float32)
    a = jnp.concatenate(
        [jnp.zeros((ANGULAR_BINS, 1), jnp.float32), aw], axis=1)
    aext = jnp.zeros((A_PAD, E_DIM), jnp.float32)
    aext = aext.at[:ANGULAR_BINS].set(a)
    aext = aext.at[ANGULAR_BINS, 0].set(1.0)
    rc2d = r_centres.astype(jnp.float32).reshape(RADIAL_BINS, 1)

    comb, hist, tad, loss, emean, div, perp, table = pl.pallas_call(
        functools.partial(_vq_kernel, nblk=nblk, n=n),
        grid=(nblk,),
        in_specs=[
            pl.BlockSpec((BLK, E_DIM), lambda i: (i, 0)),
            pl.BlockSpec((RADIAL_BINS, 1), lambda i: (0, 0)),
            pl.BlockSpec((A_PAD, E_DIM), lambda i: (0, 0)),
        ],
        out_specs=[
            pl.BlockSpec((1, BLK), lambda i: (0, i)),
            pl.BlockSpec((RADIAL_BINS, ANGULAR_BINS), lambda i: (0, 0)),
            pl.BlockSpec((1, 1), lambda i: (0, 0)),
            pl.BlockSpec((1, 1), lambda i: (0, 0)),
            pl.BlockSpec((RADIAL_BINS, ANGULAR_BINS), lambda i: (0, 0)),
            pl.BlockSpec((1, 1), lambda i: (0, 0)),
            pl.BlockSpec((1, 1), lambda i: (0, 0)),
            pl.BlockSpec((N_E, E_DIM), lambda i: (0, 0)),
        ],
        out_shape=[
            jax.ShapeDtypeStruct((1, n), jnp.int32),
            jax.ShapeDtypeStruct((RADIAL_BINS, ANGULAR_BINS), jnp.float32),
            jax.ShapeDtypeStruct((1, 1), jnp.float32),
            jax.ShapeDtypeStruct((1, 1), jnp.float32),
            jax.ShapeDtypeStruct((RADIAL_BINS, ANGULAR_BINS), jnp.float32),
            jax.ShapeDtypeStruct((1, 1), jnp.float32),
            jax.ShapeDtypeStruct((1, 1), jnp.float32),
            jax.ShapeDtypeStruct((N_E, E_DIM), jnp.float32),
        ],
    )(flat, rc2d, aext)

    info = plsc.get_sparse_core_info()
    nw = info.num_cores * info.num_subcores
    rows_per_w = n // nw
    z_q = _sc_zq_gather(rows_per_w, n, shape)(table, comb.reshape(n))
    return (loss[0, 0], z_q, perp[0, 0], div[0, 0], emean.reshape(N_E))


# speedup vs baseline: 1.0236x; 1.0236x over previous
import functools

import jax
import jax.numpy as jnp
from jax import lax
from jax.experimental import pallas as pl
from jax.experimental.pallas import tpu as pltpu
from jax.experimental.pallas import tpu_sc as plsc

N_E = 8192
E_DIM = 64
BETA = 0.25
RADIAL_BINS = 16
ANGULAR_BINS = N_E // RADIAL_BINS
MAX_RADIUS = 1.1
BLK = 3072
A_PAD = ANGULAR_BINS + 8
IDX_CHUNK = 128


def _acosh(x):
    return jnp.log(x + jnp.sqrt((x - 1.0) * (x + 1.0)))


def _vq_kernel(x_ref, rc_ref, a_ref,
               comb_ref, hist_ref, tad_ref,
               loss_ref, emean_ref, div_ref, perp_ref, *, nblk, n):
    i = pl.program_id(0)
    x = x_ref[...]

    dotm = jax.lax.dot_general(a_ref[...], x, (((1,), (1,)), ((), ())),
                               precision=jax.lax.Precision.HIGHEST,
                               preferred_element_type=jnp.float32)
    u_t = dotm[ANGULAR_BINS:ANGULAR_BINS + 1, :]
    r = _acosh(jnp.maximum(u_t, 1.01))
    dot_v = dotm[:ANGULAR_BINS]

    iota_w = jax.lax.broadcasted_iota(jnp.int32, (ANGULAR_BINS, BLK), 0)
    dval = jnp.max(dot_v, axis=0, keepdims=True)
    best_widx = jnp.min(jnp.where(dot_v == dval, iota_w, ANGULAR_BINS),
                        axis=0, keepdims=True)

    rc = jnp.clip(rc_ref[...], 0.01, MAX_RADIUS)
    dist_r = jnp.abs(r - rc)
    iota_r = jax.lax.broadcasted_iota(jnp.int32, (RADIAL_BINS, BLK), 0)
    r_sel = []
    for _ in range(3):
        v = jnp.min(dist_r, axis=0, keepdims=True)
        idx = jnp.min(jnp.where(dist_r == v, iota_r, RADIAL_BINS),
                      axis=0, keepdims=True)
        rc_v = jnp.max(jnp.where(iota_r == idx, rc + 0.0 * dist_r, -jnp.inf),
                       axis=0, keepdims=True)
        r_sel.append((rc_v, idx))
        dist_r = jnp.where(iota_r == idx, jnp.inf, dist_r)

    best_d = jnp.full((1, BLK), jnp.inf, dtype=jnp.float32)
    best_ridx = jnp.zeros((1, BLK), dtype=jnp.int32)
    best_cosh = jnp.ones((1, BLK), dtype=jnp.float32)
    for rc_v, ridx in r_sel:
        e = jnp.exp(rc_v)
        ch = 0.5 * (e + 1.0 / e)
        sh = 0.5 * (e - 1.0 / e)
        arg = jnp.maximum(u_t * ch - sh * dval, 1.0 + 1e-7)
        d = _acosh(arg)
        mask = d < best_d
        best_d = jnp.where(mask, d, best_d)
        best_ridx = jnp.where(mask, ridx, best_ridx)
        best_cosh = jnp.where(mask, ch, best_cosh)

    comb_ref[...] = best_ridx * ANGULAR_BINS + best_widx

    rx = _acosh(jnp.maximum(u_t, 1.0 + 1e-5))
    ry = _acosh(jnp.maximum(best_cosh, 1.0 + 1e-5))
    tad_part = jnp.sum(best_d + jnp.abs(rx - ry))

    oh_w = (iota_w == best_widx).astype(jnp.bfloat16)
    oh_r = (iota_r == best_ridx).astype(jnp.bfloat16)
    hist_part = jax.lax.dot_general(
        oh_r, oh_w, (((1,), (1,)), ((), ())),
        preferred_element_type=jnp.float32)

    @pl.when(i == 0)
    def _():
        hist_ref[...] = hist_part
        tad_ref[...] = tad_part.reshape(1, 1)

    @pl.when(i > 0)
    def _():
        hist_ref[...] += hist_part
        tad_ref[...] += tad_part.reshape(1, 1)

    @pl.when(i == nblk - 1)
    def _():
        e_mean = hist_ref[...] * (1.0 / n)
        emean_ref[...] = e_mean
        div = -jnp.sum(e_mean * jnp.log(e_mean + 1e-10))
        div_ref[...] = div.reshape(1, 1)
        perp_ref[...] = jnp.exp(div).reshape(1, 1)
        loss_ref[...] = (1.0 + BETA) * tad_ref[...] * (1.0 / n)


def _table_kernel(rc_ref, a_ref, tab_ref):
    i = pl.program_id(0)
    rc_all = jnp.clip(rc_ref[...], 0.01, MAX_RADIUS)
    sel = jax.lax.broadcasted_iota(jnp.int32, (RADIAL_BINS, 1), 0) == i
    rc = jnp.max(jnp.where(sel, rc_all, -jnp.inf))
    e = jnp.exp(rc)
    ch = 0.5 * (e + 1.0 / e)
    sh = 0.5 * (e - 1.0 / e)
    a = a_ref[...]
    e0 = (jax.lax.broadcasted_iota(jnp.int32, (ANGULAR_BINS, E_DIM), 1) == 0)
    tab_ref[...] = a * sh + jnp.where(e0, ch, 0.0)


def _sc_zq_gather(rows_per_w, n, out_shape):
    nchunk = rows_per_w // IDX_CHUNK
    t_dim = out_shape[1]
    planes_per_w = rows_per_w // t_dim
    mesh = plsc.VectorSubcoreMesh(core_axis_name="c", subcore_axis_name="s")

    @functools.partial(
        pl.kernel, mesh=mesh,
        compiler_params=pltpu.CompilerParams(use_tc_tiling_on_sc=False),
        out_type=jax.ShapeDtypeStruct(out_shape, jnp.float32),
        scratch_types=[
            pltpu.VMEM((nchunk, IDX_CHUNK), jnp.int32),
            pltpu.VMEM((rows_per_w, E_DIM), jnp.float32),
            pltpu.SemaphoreType.DMA,
        ],
    )
    def zq_gather(tab_hbm, comb_hbm, zq_hbm, idx_v, rows_v, sem):
        wid = lax.axis_index("s") * 2 + lax.axis_index("c")
        base = wid * rows_per_w
        for j in range(nchunk):
            pltpu.sync_copy(
                comb_hbm.at[pl.ds(base + j * IDX_CHUNK, IDX_CHUNK)],
                idx_v.at[j])
        copies = []
        for j in range(nchunk):
            copies.append(pltpu.async_copy(
                tab_hbm.at[idx_v.at[j]],
                rows_v.at[pl.ds(j * IDX_CHUNK, IDX_CHUNK)], sem))
        for c in copies:
            c.wait()
        for p in range(planes_per_w):
            pltpu.sync_copy(rows_v.at[pl.ds(p * t_dim, t_dim)],
                            zq_hbm.at[planes_per_w * wid + p])

    return zq_gather


def kernel(u_hyp, r_centres, angular_weight):
    shape = u_hyp.shape
    flat = u_hyp.reshape(-1, shape[-1]).astype(jnp.float32)
    n = flat.shape[0]
    nblk = n // BLK
    aw = angular_weight.astype(jnp.float32)
    a = jnp.concatenate(
        [jnp.zeros((ANGULAR_BINS, 1), jnp.float32), aw], axis=1)
    aext = jnp.zeros((A_PAD, E_DIM), jnp.float32)
    aext = aext.at[:ANGULAR_BINS].set(a)
    aext = aext.at[ANGULAR_BINS, 0].set(1.0)
    rc2d = r_centres.astype(jnp.float32).reshape(RADIAL_BINS, 1)

    comb, hist, tad, loss, emean, div, perp = pl.pallas_call(
        functools.partial(_vq_kernel, nblk=nblk, n=n),
        grid=(nblk,),
        in_specs=[
            pl.BlockSpec((BLK, E_DIM), lambda i: (i, 0)),
            pl.BlockSpec((RADIAL_BINS, 1), lambda i: (0, 0)),
            pl.BlockSpec((A_PAD, E_DIM), lambda i: (0, 0)),
        ],
        out_specs=[
            pl.BlockSpec((1, BLK), lambda i: (0, i)),
            pl.BlockSpec((RADIAL_BINS, ANGULAR_BINS), lambda i: (0, 0)),
            pl.BlockSpec((1, 1), lambda i: (0, 0)),
            pl.BlockSpec((1, 1), lambda i: (0, 0)),
            pl.BlockSpec((RADIAL_BINS, ANGULAR_BINS), lambda i: (0, 0)),
            pl.BlockSpec((1, 1), lambda i: (0, 0)),
            pl.BlockSpec((1, 1), lambda i: (0, 0)),
        ],
        out_shape=[
            jax.ShapeDtypeStruct((1, n), jnp.int32),
            jax.ShapeDtypeStruct((RADIAL_BINS, ANGULAR_BINS), jnp.float32),
            jax.ShapeDtypeStruct((1, 1), jnp.float32),
            jax.ShapeDtypeStruct((1, 1), jnp.float32),
            jax.ShapeDtypeStruct((RADIAL_BINS, ANGULAR_BINS), jnp.float32),
            jax.ShapeDtypeStruct((1, 1), jnp.float32),
            jax.ShapeDtypeStruct((1, 1), jnp.float32),
        ],
    )(flat, rc2d, aext)

    table = pl.pallas_call(
        _table_kernel,
        grid=(RADIAL_BINS,),
        in_specs=[
            pl.BlockSpec((RADIAL_BINS, 1), lambda i: (0, 0)),
            pl.BlockSpec((ANGULAR_BINS, E_DIM), lambda i: (0, 0)),
        ],
        out_specs=pl.BlockSpec((ANGULAR_BINS, E_DIM), lambda i: (i, 0)),
        out_shape=jax.ShapeDtypeStruct((N_E, E_DIM), jnp.float32),
    )(rc2d, a)

    info = plsc.get_sparse_core_info()
    nw = info.num_cores * info.num_subcores
    rows_per_w = n // nw
    z_q = _sc_zq_gather(rows_per_w, n, shape)(table, comb.reshape(n))
    return (loss[0, 0], z_q, perp[0, 0], div[0, 0], emean.reshape(N_E))
